# baseline (speedup 1.0000x reference)
import jax
import jax.numpy as jnp
from jax.experimental import pallas as pl

O_N = 10000
NUM_OBJS_P1 = 180
NIMG = 8
EMB = 128
GDIM = 128
HID = 512
NOISE_DIM = 64

KPAD = 184
CROWS = KPAD + NIMG + NOISE_DIM
BLK = 5000
NB = O_N // BLK

_HI = jax.lax.Precision.HIGHEST


def _prep_kernel(objs_ref, oti_ref, obj_emb_ref, gconv_W_ref, gconv_b_ref,
                 att_W_ref, W1a_ref, W1b_ref, W1c_ref, b1_ref,
                 CChi_ref, CClo_ref):
    objs_l = objs_ref[...]
    oti_l = oti_ref[...]
    ohT_obj = (jax.lax.broadcasted_iota(jnp.int32, (KPAD, O_N), 0)
               == objs_l).astype(jnp.bfloat16)
    ohT_img = (jax.lax.broadcasted_iota(jnp.int32, (NIMG, O_N), 0)
               == oti_l).astype(jnp.bfloat16)
    histT = jax.lax.dot_general(ohT_obj, ohT_img, (((1,), (1,)), ((), ())),
                                preferred_element_type=jnp.float32)
    table_g = jnp.dot(obj_emb_ref[...], gconv_W_ref[...],
                      preferred_element_type=jnp.float32) + gconv_b_ref[...]
    table_a = jnp.dot(table_g, att_W_ref[...],
                      preferred_element_type=jnp.float32)
    counts = jax.lax.dot_general(
        histT, jnp.ones((KPAD, 1), jnp.float32),
        (((0,), (0,)), ((), ())), preferred_element_type=jnp.float32)
    counts = jnp.where(counts > 0.0, counts, 1.0)
    gc = jax.lax.dot_general(
        histT, table_a, (((0,), (0,)), ((), ())), precision=_HI,
        preferred_element_type=jnp.float32) / counts
    tg = jnp.tanh(gc)
    sig = jax.nn.sigmoid(jax.lax.dot_general(
        table_g, tg, (((1,), (1,)), ((), ())), precision=_HI,
        preferred_element_type=jnp.float32))
    w = histT * sig
    rep = jax.lax.dot_general(
        w, table_g, (((0,), (0,)), ((), ())), precision=_HI,
        preferred_element_type=jnp.float32)
    A = jnp.dot(table_g, W1a_ref[...], preferred_element_type=jnp.float32)
    Brep = jnp.dot(rep, W1b_ref[...],
                   preferred_element_type=jnp.float32) + b1_ref[...]
    AB = jnp.concatenate([A, Brep], axis=0)
    AB_hi = AB.astype(jnp.bfloat16)
    AB_lo = (AB - AB_hi.astype(jnp.float32)).astype(jnp.bfloat16)
    W1c_hi = W1c_ref[...].astype(jnp.bfloat16)
    CChi_ref[...] = jnp.concatenate([AB_hi, W1c_hi], axis=0)
    CClo_ref[...] = jnp.concatenate(
        [AB_lo, jnp.zeros((NOISE_DIM, HID), jnp.bfloat16)], axis=0)


def _main_kernel(objs_ref, oti_ref, noiseT_ref, CChi_ref, CClo_ref, W2_ref,
                 b2_ref, out_ref):
    objs_l = objs_ref[0]
    oti_l = oti_ref[0]
    ohT_obj = (jax.lax.broadcasted_iota(jnp.int32, (KPAD, BLK), 0)
               == objs_l).astype(jnp.bfloat16)
    ohT_img = (jax.lax.broadcasted_iota(jnp.int32, (NIMG, BLK), 0)
               == oti_l).astype(jnp.bfloat16)
    M = jnp.concatenate([ohT_obj, ohT_img, noiseT_ref[0]], axis=0)
    hx = jax.lax.dot_general(M, CChi_ref[...], (((0,), (0,)), ((), ())),
                             preferred_element_type=jnp.float32)
    hy = jax.lax.dot_general(M, CClo_ref[...], (((0,), (0,)), ((), ())),
                             preferred_element_type=jnp.float32)
    h = jax.nn.relu(hx + hy)
    out_ref[...] = jnp.dot(h.astype(jnp.bfloat16), W2_ref[...],
                           preferred_element_type=jnp.float32) + b2_ref[...]


@jax.jit
def _run(objs, noise, obj_to_img, obj_emb, gconv_W, gconv_b, att_W,
         box_W1, box_b1, box_W2, box_b2):
    objs_r = objs.astype(jnp.int32).reshape(1, O_N)
    oti_r = obj_to_img.astype(jnp.int32).reshape(1, O_N)
    obj_emb_p = jnp.pad(obj_emb, ((0, KPAD - NUM_OBJS_P1), (0, 0)))
    noiseT = noise.astype(jnp.bfloat16).reshape(NB, BLK, NOISE_DIM).swapaxes(1, 2)
    W2_bf = box_W2.astype(jnp.bfloat16)

    def full(shape, idx=None):
        if idx is None:
            idx = tuple(0 for _ in shape)
        return pl.BlockSpec(shape, lambda b, _i=idx: _i)

    CChi, CClo = pl.pallas_call(
        _prep_kernel,
        grid=(1,),
        in_specs=[
            full((1, O_N)), full((1, O_N)),
            full((KPAD, EMB)), full((EMB, GDIM)), full((1, GDIM)),
            full((GDIM, GDIM)),
            full((GDIM, HID)),
            full((GDIM, HID), (1, 0)),
            full((NOISE_DIM, HID), (4, 0)),
            full((1, HID)),
        ],
        out_specs=[full((CROWS, HID)), full((CROWS, HID))],
        out_shape=[jax.ShapeDtypeStruct((CROWS, HID), jnp.bfloat16),
                   jax.ShapeDtypeStruct((CROWS, HID), jnp.bfloat16)],
    )(objs_r, oti_r, obj_emb_p, gconv_W, gconv_b.reshape(1, GDIM), att_W,
      box_W1, box_W1, box_W1, box_b1.reshape(1, HID))

    objs_b = objs.astype(jnp.int32).reshape(NB, 1, BLK)
    oti_b = obj_to_img.astype(jnp.int32).reshape(NB, 1, BLK)
    out = pl.pallas_call(
        _main_kernel,
        grid=(NB,),
        in_specs=[
            pl.BlockSpec((1, 1, BLK), lambda b: (b, 0, 0)),
            pl.BlockSpec((1, 1, BLK), lambda b: (b, 0, 0)),
            pl.BlockSpec((1, NOISE_DIM, BLK), lambda b: (b, 0, 0)),
            full((CROWS, HID)), full((CROWS, HID)),
            full((HID, 4)), full((1, 4)),
        ],
        out_specs=pl.BlockSpec((BLK, 4), lambda b: (b, 0)),
        out_shape=jax.ShapeDtypeStruct((O_N, 4), jnp.float32),
    )(objs_b, oti_b, noiseT, CChi, CClo, W2_bf, box_b2.reshape(1, 4))

    return out


def kernel(objs, triples, noise, obj_to_img, obj_emb, pred_emb, gconv_W,
           gconv_b, att_W, box_W1, box_b1, box_W2, box_b2):
    del triples, pred_emb
    return _run(objs, noise, obj_to_img, obj_emb, gconv_W, gconv_b, att_W,
                box_W1, box_b1, box_W2, box_b2)


# speedup vs baseline: 1.0112x; 1.0112x over previous
import jax
import jax.numpy as jnp
from jax.experimental import pallas as pl
from jax.experimental.pallas import tpu as pltpu

O_N = 10000
NUM_OBJS_P1 = 180
NIMG = 8
EMB = 128
GDIM = 128
HID = 512
NOISE_DIM = 64

KPAD = 184
CROWS = KPAD + NIMG + NOISE_DIM
BLK = 5000
NB = O_N // BLK

_HI = jax.lax.Precision.HIGHEST


def _fused_kernel(objs_ref, oti_ref, objs_b_ref, oti_b_ref, noiseT_ref,
                  obj_emb_ref, gconv_W_ref, gconv_b_ref, att_W_ref,
                  W1a_ref, W1b_ref, W1c_ref, b1_ref, W2_ref, b2_ref,
                  out_ref, CChi_s, CClo_s):
    s = pl.program_id(0)

    @pl.when(s == 0)
    def _prep():
        objs_l = objs_ref[...]
        oti_l = oti_ref[...]
        ohT_obj = (jax.lax.broadcasted_iota(jnp.int32, (KPAD, O_N), 0)
                   == objs_l).astype(jnp.bfloat16)
        ohT_img = (jax.lax.broadcasted_iota(jnp.int32, (NIMG, O_N), 0)
                   == oti_l).astype(jnp.bfloat16)
        histT = jax.lax.dot_general(
            ohT_obj, ohT_img, (((1,), (1,)), ((), ())),
            preferred_element_type=jnp.float32)
        table_g = jnp.dot(obj_emb_ref[...], gconv_W_ref[...],
                          preferred_element_type=jnp.float32) + gconv_b_ref[...]
        table_a = jnp.dot(table_g, att_W_ref[...],
                          preferred_element_type=jnp.float32)
        counts = jax.lax.dot_general(
            histT, jnp.ones((KPAD, 1), jnp.float32),
            (((0,), (0,)), ((), ())), preferred_element_type=jnp.float32)
        counts = jnp.where(counts > 0.0, counts, 1.0)
        gc = jax.lax.dot_general(
            histT, table_a, (((0,), (0,)), ((), ())), precision=_HI,
            preferred_element_type=jnp.float32) / counts
        tg = jnp.tanh(gc)
        sig = jax.nn.sigmoid(jax.lax.dot_general(
            table_g, tg, (((1,), (1,)), ((), ())), precision=_HI,
            preferred_element_type=jnp.float32))
        w = histT * sig
        rep = jax.lax.dot_general(
            w, table_g, (((0,), (0,)), ((), ())), precision=_HI,
            preferred_element_type=jnp.float32)
        A = jnp.dot(table_g, W1a_ref[...],
                    preferred_element_type=jnp.float32)
        Brep = jnp.dot(rep, W1b_ref[...],
                       preferred_element_type=jnp.float32) + b1_ref[...]
        AB = jnp.concatenate([A, Brep], axis=0)
        AB_hi = AB.astype(jnp.bfloat16)
        AB_lo = (AB - AB_hi.astype(jnp.float32)).astype(jnp.bfloat16)
        W1c_hi = W1c_ref[...].astype(jnp.bfloat16)
        CChi_s[...] = jnp.concatenate([AB_hi, W1c_hi], axis=0)
        CClo_s[...] = jnp.concatenate(
            [AB_lo, jnp.zeros((NOISE_DIM, HID), jnp.bfloat16)], axis=0)

    @pl.when(s > 0)
    def _main():
        objs_l = objs_b_ref[0]
        oti_l = oti_b_ref[0]
        ohT_obj = (jax.lax.broadcasted_iota(jnp.int32, (KPAD, BLK), 0)
                   == objs_l).astype(jnp.bfloat16)
        ohT_img = (jax.lax.broadcasted_iota(jnp.int32, (NIMG, BLK), 0)
                   == oti_l).astype(jnp.bfloat16)
        M = jnp.concatenate([ohT_obj, ohT_img, noiseT_ref[0]], axis=0)
        hx = jax.lax.dot_general(M, CChi_s[...], (((0,), (0,)), ((), ())),
                                 preferred_element_type=jnp.float32)
        hy = jax.lax.dot_general(M, CClo_s[...], (((0,), (0,)), ((), ())),
                                 preferred_element_type=jnp.float32)
        h = jax.nn.relu(hx + hy)
        out_ref[...] = jnp.dot(h.astype(jnp.bfloat16), W2_ref[...],
                               preferred_element_type=jnp.float32) + b2_ref[...]


@jax.jit
def _run(objs, noise, obj_to_img, obj_emb, gconv_W, gconv_b, att_W,
         box_W1, box_b1, box_W2, box_b2):
    objs_r = objs.astype(jnp.int32).reshape(1, O_N)
    oti_r = obj_to_img.astype(jnp.int32).reshape(1, O_N)
    objs_b = objs.astype(jnp.int32).reshape(NB, 1, BLK)
    oti_b = obj_to_img.astype(jnp.int32).reshape(NB, 1, BLK)
    obj_emb_p = jnp.pad(obj_emb, ((0, KPAD - NUM_OBJS_P1), (0, 0)))
    noiseT = noise.astype(jnp.bfloat16).reshape(NB, BLK, NOISE_DIM).swapaxes(1, 2)
    W2_bf = box_W2.astype(jnp.bfloat16)

    def full(shape, idx=None):
        if idx is None:
            idx = tuple(0 for _ in shape)
        return pl.BlockSpec(shape, lambda s, _i=idx: _i)

    def blk3(n1, n2):
        return pl.BlockSpec((1, n1, n2),
                            lambda s: (jnp.maximum(s - 1, 0), 0, 0))

    out = pl.pallas_call(
        _fused_kernel,
        grid=(NB + 1,),
        in_specs=[
            full((1, O_N)), full((1, O_N)),
            blk3(1, BLK), blk3(1, BLK), blk3(NOISE_DIM, BLK),
            full((KPAD, EMB)), full((EMB, GDIM)), full((1, GDIM)),
            full((GDIM, GDIM)),
            full((GDIM, HID)),
            full((GDIM, HID), (1, 0)),
            full((NOISE_DIM, HID), (4, 0)),
            full((1, HID)),
            full((HID, 4)), full((1, 4)),
        ],
        out_specs=pl.BlockSpec((BLK, 4), lambda s: (jnp.maximum(s - 1, 0), 0)),
        out_shape=jax.ShapeDtypeStruct((O_N, 4), jnp.float32),
        scratch_shapes=[pltpu.VMEM((CROWS, HID), jnp.bfloat16),
                        pltpu.VMEM((CROWS, HID), jnp.bfloat16)],
    )(objs_r, oti_r, objs_b, oti_b, noiseT, obj_emb_p, gconv_W,
      gconv_b.reshape(1, GDIM), att_W, box_W1, box_W1, box_W1,
      box_b1.reshape(1, HID), W2_bf, box_b2.reshape(1, 4))

    return out


def kernel(objs, triples, noise, obj_to_img, obj_emb, pred_emb, gconv_W,
           gconv_b, att_W, box_W1, box_b1, box_W2, box_b2):
    del triples, pred_emb
    return _run(objs, noise, obj_to_img, obj_emb, gconv_W, gconv_b, att_W,
                box_W1, box_b1, box_W2, box_b2)


# speedup vs baseline: 1.1708x; 1.1577x over previous
import jax
import jax.numpy as jnp
from jax.experimental import pallas as pl

O_N = 10000
NUM_OBJS_P1 = 180
NIMG = 8
EMB = 128
GDIM = 128
HID = 512
NOISE_DIM = 64

KPAD = 184
CROWS = KPAD + NIMG + NOISE_DIM

_HI = jax.lax.Precision.HIGHEST


def _mono_kernel(objs_ref, oti_ref, noiseT_ref, obj_emb_ref, gconv_W_ref,
                 gconv_b_ref, att_W_ref, W1a_ref, W1b_ref, W1c_ref, b1_ref,
                 W2_ref, b2_ref, out_ref):
    objs_l = objs_ref[...]
    oti_l = oti_ref[...]
    ohT_obj = (jax.lax.broadcasted_iota(jnp.int32, (KPAD, O_N), 0)
               == objs_l).astype(jnp.bfloat16)
    ohT_img = (jax.lax.broadcasted_iota(jnp.int32, (NIMG, O_N), 0)
               == oti_l).astype(jnp.bfloat16)
    histT = jax.lax.dot_general(ohT_obj, ohT_img, (((1,), (1,)), ((), ())),
                                preferred_element_type=jnp.float32)
    table_g = jnp.dot(obj_emb_ref[...], gconv_W_ref[...],
                      preferred_element_type=jnp.float32) + gconv_b_ref[...]
    table_a = jnp.dot(table_g, att_W_ref[...],
                      preferred_element_type=jnp.float32)
    counts = jax.lax.dot_general(
        histT, jnp.ones((KPAD, 1), jnp.float32),
        (((0,), (0,)), ((), ())), preferred_element_type=jnp.float32)
    counts = jnp.where(counts > 0.0, counts, 1.0)
    gc = jax.lax.dot_general(
        histT, table_a, (((0,), (0,)), ((), ())), precision=_HI,
        preferred_element_type=jnp.float32) / counts
    tg = jnp.tanh(gc)
    sig = jax.nn.sigmoid(jax.lax.dot_general(
        table_g, tg, (((1,), (1,)), ((), ())), precision=_HI,
        preferred_element_type=jnp.float32))
    w = histT * sig
    rep = jax.lax.dot_general(
        w, table_g, (((0,), (0,)), ((), ())), precision=_HI,
        preferred_element_type=jnp.float32)
    A = jnp.dot(table_g, W1a_ref[...], preferred_element_type=jnp.float32)
    Brep = jnp.dot(rep, W1b_ref[...],
                   preferred_element_type=jnp.float32) + b1_ref[...]
    AB = jnp.concatenate([A, Brep], axis=0)
    AB_hi = AB.astype(jnp.bfloat16)
    AB_lo = (AB - AB_hi.astype(jnp.float32)).astype(jnp.bfloat16)
    W1c_hi = W1c_ref[...].astype(jnp.bfloat16)
    CC2 = jnp.concatenate(
        [AB_hi, W1c_hi, AB_lo, jnp.zeros((NOISE_DIM, HID), jnp.bfloat16)],
        axis=0)
    noiseT = noiseT_ref[...]
    M2 = jnp.concatenate(
        [ohT_obj, ohT_img, noiseT, ohT_obj, ohT_img, noiseT], axis=0)
    h = jax.nn.relu(jax.lax.dot_general(
        M2, CC2, (((0,), (0,)), ((), ())),
        preferred_element_type=jnp.float32))
    out_ref[...] = jnp.dot(h.astype(jnp.bfloat16), W2_ref[...],
                           preferred_element_type=jnp.float32) + b2_ref[...]


@jax.jit
def _run(objs, noise, obj_to_img, obj_emb, gconv_W, gconv_b, att_W,
         box_W1, box_b1, box_W2, box_b2):
    objs_r = objs.astype(jnp.int32).reshape(1, O_N)
    oti_r = obj_to_img.astype(jnp.int32).reshape(1, O_N)
    obj_emb_p = jnp.pad(obj_emb, ((0, KPAD - NUM_OBJS_P1), (0, 0)))
    noiseT = noise.astype(jnp.bfloat16).T
    W2_bf = box_W2.astype(jnp.bfloat16)

    def full(shape, idx=None):
        if idx is None:
            idx = tuple(0 for _ in shape)
        return pl.BlockSpec(shape, lambda s, _i=idx: _i)

    out = pl.pallas_call(
        _mono_kernel,
        grid=(1,),
        in_specs=[
            full((1, O_N)), full((1, O_N)), full((NOISE_DIM, O_N)),
            full((KPAD, EMB)), full((EMB, GDIM)), full((1, GDIM)),
            full((GDIM, GDIM)),
            full((GDIM, HID)),
            full((GDIM, HID), (1, 0)),
            full((NOISE_DIM, HID), (4, 0)),
            full((1, HID)),
            full((HID, 4)), full((1, 4)),
        ],
        out_specs=full((O_N, 4)),
        out_shape=jax.ShapeDtypeStruct((O_N, 4), jnp.float32),
    )(objs_r, oti_r, noiseT, obj_emb_p, gconv_W, gconv_b.reshape(1, GDIM),
      att_W, box_W1, box_W1, box_W1, box_b1.reshape(1, HID), W2_bf,
      box_b2.reshape(1, 4))

    return out


def kernel(objs, triples, noise, obj_to_img, obj_emb, pred_emb, gconv_W,
           gconv_b, att_W, box_W1, box_b1, box_W2, box_b2):
    del triples, pred_emb
    return _run(objs, noise, obj_to_img, obj_emb, gconv_W, gconv_b, att_W,
                box_W1, box_b1, box_W2, box_b2)
